# Initial kernel scaffold; baseline (speedup 1.0000x reference)
import functools

import jax
import jax.numpy as jnp
from jax import lax
from jax.experimental import pallas as pl
from jax.experimental.pallas import tpu as pltpu
from jax.experimental.pallas import tpu_sc as plsc

B, S, NPG = 10, 8, 5000
N = B * NPG
E = 800000
DIN, DM, DOUT = 3, 64, 2
HALF = DM // 2

NC, NS = 2, 16
EPAD = 819200
ROWS = EPAD // 128
ROWS_PC = ROWS // NC
ROWS_PS = ROWS // NS
MACRO = 16
NMACRO = ROWS_PS // MACRO

ACC_ROWS = 50016
ROWS_PS_N = N // NS
CHUNK = 625

DEG_LEN = 50176
DEG_PS = DEG_LEN // NS


def _mesh():
    return plsc.VectorSubcoreMesh(core_axis_name="c", subcore_axis_name="s")


def _deg_body(dst_hbm, out_hbm, dst_v, ones_v, stage_v, acc):
    cid = lax.axis_index("c")
    sid = lax.axis_index("s")

    for i in range(8):
        ones_v[pl.ds(i * 16, 16)] = jnp.full((16,), 1.0, jnp.float32)

    def _zero(i, _):
        stage_v[pl.ds(i * 16, 16)] = jnp.zeros((16,), jnp.float32)
        return 0

    lax.fori_loop(0, DEG_PS // 16, _zero, 0)
    pltpu.sync_copy(stage_v, acc.at[pl.ds(sid * DEG_PS, DEG_PS)])
    plsc.subcore_barrier()

    row0 = cid * ROWS_PC + sid * (ROWS_PC // NS)
    pltpu.sync_copy(dst_hbm.at[pl.ds(row0, ROWS_PC // NS)], dst_v)

    def _scat(i, _):
        pltpu.sync_copy(ones_v, acc.at[dst_v.at[i]], add=True)
        return 0

    lax.fori_loop(0, ROWS_PC // NS, _scat, 0)
    plsc.subcore_barrier()

    pltpu.sync_copy(acc.at[pl.ds(sid * DEG_PS, DEG_PS)], stage_v)
    pltpu.sync_copy(stage_v, out_hbm.at[pl.ds(cid * DEG_LEN + sid * DEG_PS, DEG_PS)])


_deg_call = pl.kernel(
    _deg_body,
    out_type=jax.ShapeDtypeStruct((NC * DEG_LEN,), jnp.float32),
    mesh=_mesh(),
    scratch_types=[
        pltpu.VMEM((ROWS_PC // NS, 128), jnp.int32),
        pltpu.VMEM((128,), jnp.float32),
        pltpu.VMEM((DEG_PS,), jnp.float32),
        pltpu.VMEM_SHARED((DEG_LEN,), jnp.float32),
    ],
    name="sc_degree",
)


def _agg_body(src_hbm, dst_hbm, tab_hbm, out_hbm,
              src_v, dst_v, msg0, msg1, stage_v, sem0, sem1, acc):
    cid = lax.axis_index("c")
    sid = lax.axis_index("s")

    for t in range(ROWS_PS_N // CHUNK):
        r0 = sid * ROWS_PS_N + t * CHUNK
        pltpu.sync_copy(tab_hbm.at[pl.ds(cid * N + r0, CHUNK)], stage_v)
        pltpu.sync_copy(stage_v, acc.at[pl.ds(r0, CHUNK)])
    plsc.subcore_barrier()

    row_base = sid * ROWS_PS
    msgs = (msg0, msg1)
    sems = (sem0, sem1)

    def _macro(m, _):
        r = row_base + m * MACRO
        pltpu.sync_copy(src_hbm.at[pl.ds(cid * ROWS + r, MACRO)], src_v)
        pltpu.sync_copy(dst_hbm.at[pl.ds(r, MACRO)], dst_v)
        cp = pltpu.async_copy(tab_hbm.at[src_v.at[0]], msgs[0], sems[0])
        for j in range(MACRO):
            cp.wait()
            if j + 1 < MACRO:
                cp = pltpu.async_copy(
                    tab_hbm.at[src_v.at[j + 1]], msgs[(j + 1) % 2], sems[(j + 1) % 2])
            pltpu.sync_copy(msgs[j % 2], acc.at[dst_v.at[j]], add=True)
        return 0

    lax.fori_loop(0, NMACRO, _macro, 0)
    plsc.subcore_barrier()

    for t in range(ROWS_PS_N // CHUNK):
        r0 = sid * ROWS_PS_N + t * CHUNK
        pltpu.sync_copy(acc.at[pl.ds(r0, CHUNK)], stage_v)
        pltpu.sync_copy(stage_v, out_hbm.at[pl.ds(cid * N + r0, CHUNK)])


_agg_call = pl.kernel(
    _agg_body,
    out_type=jax.ShapeDtypeStruct((NC * N, HALF), jnp.float32),
    mesh=_mesh(),
    scratch_types=[
        pltpu.VMEM((MACRO, 128), jnp.int32),
        pltpu.VMEM((MACRO, 128), jnp.int32),
        pltpu.VMEM((128, HALF), jnp.float32),
        pltpu.VMEM((128, HALF), jnp.float32),
        pltpu.VMEM((CHUNK, HALF), jnp.float32),
        pltpu.SemaphoreType.DMA,
        pltpu.SemaphoreType.DMA,
        pltpu.VMEM_SHARED((ACC_ROWS, HALF), jnp.float32),
    ],
    name="sc_edge_agg",
)


def _mean_body(x_ref, o_ref):
    o_ref[...] = jnp.mean(x_ref[...], axis=1)


def _tc_mean(xr):
    return pl.pallas_call(
        _mean_body,
        grid=(B,),
        in_specs=[pl.BlockSpec((1, S, NPG * DIN), lambda b: (b, 0, 0))],
        out_specs=pl.BlockSpec((1, NPG * DIN), lambda b: (b, 0)),
        out_shape=jax.ShapeDtypeStruct((B, NPG * DIN), jnp.float32),
    )(xr)


def _prep1_body(xm_ref, da_ref, db_ref, w1_ref, hws_ref, dinv_ref):
    deg = da_ref[...] + db_ref[...] + 1.0
    dinv = lax.rsqrt(deg)
    hw = jnp.dot(xm_ref[...], w1_ref[...], preferred_element_type=jnp.float32)
    hws = hw * dinv
    hws_ref[0] = hws[:, :HALF]
    hws_ref[1] = hws[:, HALF:]
    dinv_ref[...] = dinv


def _tc_prep1(xm, dega, degb, W1):
    blk = NPG
    nblk = N // blk
    return pl.pallas_call(
        _prep1_body,
        grid=(nblk,),
        in_specs=[
            pl.BlockSpec((blk, DIN), lambda i: (i, 0)),
            pl.BlockSpec((blk, 1), lambda i: (i, 0)),
            pl.BlockSpec((blk, 1), lambda i: (i, 0)),
            pl.BlockSpec((DIN, DM), lambda i: (0, 0)),
        ],
        out_specs=[
            pl.BlockSpec((NC, blk, HALF), lambda i: (0, i, 0)),
            pl.BlockSpec((blk, 1), lambda i: (i, 0)),
        ],
        out_shape=[
            jax.ShapeDtypeStruct((NC, N, HALF), jnp.float32),
            jax.ShapeDtypeStruct((N, 1), jnp.float32),
        ],
    )(xm, dega, degb, W1)


def _prep2_body(agg_ref, dinv_ref, b1_ref, w2_ref, hws_ref):
    agg = jnp.concatenate([agg_ref[0], agg_ref[1]], axis=1)
    dinv = dinv_ref[...]
    h = jnp.maximum(agg * dinv + b1_ref[...], 0.0)
    hw = jnp.dot(h, w2_ref[...], preferred_element_type=jnp.float32)
    hws = hw * dinv
    hws_ref[0] = hws[:, :HALF]
    hws_ref[1] = hws[:, HALF:]


def _tc_prep2(agg1, dinv, b1, W2):
    blk = NPG
    nblk = N // blk
    return pl.pallas_call(
        _prep2_body,
        grid=(nblk,),
        in_specs=[
            pl.BlockSpec((NC, blk, HALF), lambda i: (0, i, 0)),
            pl.BlockSpec((blk, 1), lambda i: (i, 0)),
            pl.BlockSpec((1, DM), lambda i: (0, 0)),
            pl.BlockSpec((DM, DM), lambda i: (0, 0)),
        ],
        out_specs=pl.BlockSpec((NC, blk, HALF), lambda i: (0, i, 0)),
        out_shape=jax.ShapeDtypeStruct((NC, N, HALF), jnp.float32),
    )(agg1, dinv, b1, W2)


def _final_body(agg_ref, dinv_ref, b2_ref, wh_ref, bh_ref, o_ref):
    agg = jnp.concatenate([agg_ref[0, 0], agg_ref[1, 0]], axis=1)
    h = jnp.maximum(agg * dinv_ref[0] + b2_ref[...], 0.0)
    pooled = jnp.mean(h, axis=0, keepdims=True)
    o_ref[...] = (
        jnp.dot(pooled, wh_ref[...], preferred_element_type=jnp.float32)
        + bh_ref[...]
    )


def _tc_final(agg2, dinv, b2, Wh, bh):
    return pl.pallas_call(
        _final_body,
        grid=(B,),
        in_specs=[
            pl.BlockSpec((NC, 1, NPG, HALF), lambda b: (0, b, 0, 0)),
            pl.BlockSpec((1, NPG, 1), lambda b: (b, 0, 0)),
            pl.BlockSpec((1, DM), lambda b: (0, 0)),
            pl.BlockSpec((DM, DOUT), lambda b: (0, 0)),
            pl.BlockSpec((1, DOUT), lambda b: (0, 0)),
        ],
        out_specs=pl.BlockSpec((1, DOUT), lambda b: (b, 0)),
        out_shape=jax.ShapeDtypeStruct((B, DOUT), jnp.float32),
    )(agg2, dinv, b2, Wh, bh)


@jax.jit
def kernel(x, edge_index, W1, b1, W2, b2, Wh, bh):
    src = edge_index[0]
    dst = edge_index[1]
    npad = EPAD - E
    srcp = jnp.concatenate([src, jnp.zeros((npad,), src.dtype)])
    dstp = jnp.concatenate([dst, jnp.full((npad,), N, dst.dtype)])
    src2 = jnp.concatenate([srcp, srcp + N]).reshape(NC * ROWS, 128)
    dst2 = dstp.reshape(ROWS, 128)

    xr = x.reshape(B, S, NPG * DIN)
    xm = _tc_mean(xr).reshape(N, DIN)

    degp = _deg_call(dst2)
    dega = degp[:N].reshape(N, 1)
    degb = degp[DEG_LEN:DEG_LEN + N].reshape(N, 1)

    hws1, dinv = _tc_prep1(xm, dega, degb, W1)
    agg1 = _agg_call(src2, dst2, hws1.reshape(NC * N, HALF))
    hws2 = _tc_prep2(agg1.reshape(NC, N, HALF), dinv, b1.reshape(1, DM), W2)
    agg2 = _agg_call(src2, dst2, hws2.reshape(NC, N, HALF).reshape(NC * N, HALF))
    return _tc_final(
        agg2.reshape(NC, B, NPG, HALF),
        dinv.reshape(B, NPG, 1),
        b2.reshape(1, DM),
        Wh,
        bh.reshape(1, DOUT),
    )


# speedup vs baseline: 11.4057x; 11.4057x over previous
import jax
import jax.numpy as jnp
from jax import lax
from jax.experimental import pallas as pl
from jax.experimental.pallas import tpu as pltpu
from jax.experimental.pallas import tpu_sc as plsc

B, S, NPG = 10, 8, 5000
N = B * NPG
E = 800000
DIN, DM, DOUT = 3, 64, 2
QC = 16

NC, NS = 2, 16
EPAD = 819200
ROWS = EPAD // 128
ROWS_PC = ROWS // NC
ROWS_PS = ROWS // NS
MACRO = 16
NMACRO = ROWS_PS // MACRO

NPAD = 51200
SUBQ = NPAD // NS
CHUNK = 640

DEG_LEN = 50176
DEG_PS = DEG_LEN // NS


def _mesh():
    return plsc.VectorSubcoreMesh(core_axis_name="c", subcore_axis_name="s")


def _deg_body(dst_hbm, out_hbm, dst_v, ones_v, stage_v, acc):
    cid = lax.axis_index("c")
    sid = lax.axis_index("s")

    for i in range(8):
        ones_v[pl.ds(i * 16, 16)] = jnp.full((16,), 1.0, jnp.float32)

    def _zero(i, _):
        stage_v[pl.ds(i * 16, 16)] = jnp.zeros((16,), jnp.float32)
        return 0

    lax.fori_loop(0, DEG_PS // 16, _zero, 0)
    pltpu.sync_copy(stage_v, acc.at[pl.ds(sid * DEG_PS, DEG_PS)])
    plsc.subcore_barrier()

    row0 = cid * ROWS_PC + sid * (ROWS_PC // NS)
    pltpu.sync_copy(dst_hbm.at[pl.ds(row0, ROWS_PC // NS)], dst_v)

    def _scat(i, _):
        pltpu.sync_copy(ones_v, acc.at[dst_v.at[i]], add=True)
        return 0

    lax.fori_loop(0, ROWS_PC // NS, _scat, 0)
    plsc.subcore_barrier()

    pltpu.sync_copy(acc.at[pl.ds(sid * DEG_PS, DEG_PS)], stage_v)
    pltpu.sync_copy(stage_v, out_hbm.at[pl.ds(cid * DEG_LEN + sid * DEG_PS, DEG_PS)])


_deg_call = pl.kernel(
    _deg_body,
    out_type=jax.ShapeDtypeStruct((NC * DEG_LEN,), jnp.float32),
    mesh=_mesh(),
    scratch_types=[
        pltpu.VMEM((ROWS_PC // NS, 128), jnp.int32),
        pltpu.VMEM((128,), jnp.float32),
        pltpu.VMEM((DEG_PS,), jnp.float32),
        pltpu.VMEM_SHARED((DEG_LEN,), jnp.float32),
    ],
    compiler_params=pltpu.CompilerParams(use_tc_tiling_on_sc=False),
    name="sc_degree",
)


def _agg_body(src_hbm, dst_hbm, tab_hbm, out_hbm,
              src_v, dst_v, msg0, msg1, stage_v, sem0, sem1, acc):
    cid = lax.axis_index("c")
    sid = lax.axis_index("s")

    for t in range(SUBQ // CHUNK):
        r0 = sid * SUBQ + t * CHUNK
        pltpu.sync_copy(tab_hbm.at[pl.ds(cid * NPAD + r0, CHUNK)], stage_v)
        pltpu.sync_copy(stage_v, acc.at[pl.ds(r0, CHUNK)])
    plsc.subcore_barrier()

    row_base = sid * ROWS_PS
    msgs = (msg0, msg1)
    sems = (sem0, sem1)

    def _macro(m, _):
        r = row_base + m * MACRO
        pltpu.sync_copy(src_hbm.at[pl.ds(cid * ROWS + r, MACRO)], src_v)
        pltpu.sync_copy(dst_hbm.at[pl.ds(r, MACRO)], dst_v)
        cp = pltpu.async_copy(tab_hbm.at[src_v.at[0]], msgs[0], sems[0])
        for j in range(MACRO):
            cp.wait()
            if j + 1 < MACRO:
                cp = pltpu.async_copy(
                    tab_hbm.at[src_v.at[j + 1]], msgs[(j + 1) % 2], sems[(j + 1) % 2])
            pltpu.sync_copy(msgs[j % 2], acc.at[dst_v.at[j]], add=True)
        return 0

    lax.fori_loop(0, NMACRO, _macro, 0)
    plsc.subcore_barrier()

    for t in range(SUBQ // CHUNK):
        r0 = sid * SUBQ + t * CHUNK
        pltpu.sync_copy(acc.at[pl.ds(r0, CHUNK)], stage_v)
        pltpu.sync_copy(stage_v, out_hbm.at[pl.ds(cid * NPAD + r0, CHUNK)])


_agg_call = pl.kernel(
    _agg_body,
    out_type=jax.ShapeDtypeStruct((NC * NPAD, QC), jnp.float32),
    mesh=_mesh(),
    scratch_types=[
        pltpu.VMEM((MACRO, 128), jnp.int32),
        pltpu.VMEM((MACRO, 128), jnp.int32),
        pltpu.VMEM((128, QC), jnp.float32),
        pltpu.VMEM((128, QC), jnp.float32),
        pltpu.VMEM((CHUNK, QC), jnp.float32),
        pltpu.SemaphoreType.DMA,
        pltpu.SemaphoreType.DMA,
        pltpu.VMEM_SHARED((NPAD, QC), jnp.float32),
    ],
    compiler_params=pltpu.CompilerParams(use_tc_tiling_on_sc=False),
    name="sc_edge_agg",
)


def _mean_body(x_ref, o_ref):
    o_ref[...] = jnp.mean(x_ref[...], axis=1)


def _tc_mean(xr):
    return pl.pallas_call(
        _mean_body,
        out_shape=jax.ShapeDtypeStruct((B, NPG * DIN), jnp.float32),
    )(xr)


def _split_quarters(hws, a_ref, b_ref):
    a_ref[0] = hws[:, 0 * QC:1 * QC]
    a_ref[1] = hws[:, 1 * QC:2 * QC]
    b_ref[0] = hws[:, 2 * QC:3 * QC]
    b_ref[1] = hws[:, 3 * QC:4 * QC]


def _prep1_body(xm_ref, da_ref, db_ref, w1_ref, a_ref, b_ref, dinv_ref):
    deg = da_ref[...] + db_ref[...] + 1.0
    dinv = lax.rsqrt(deg)
    hw = jnp.dot(xm_ref[...], w1_ref[...], preferred_element_type=jnp.float32)
    _split_quarters(hw * dinv, a_ref, b_ref)
    dinv_ref[...] = dinv


def _tc_prep1(xm, dega, degb, W1):
    blk = NPG
    nblk = N // blk
    qspec = pl.BlockSpec((NC, blk, QC), lambda i: (0, i, 0))
    qshape = jax.ShapeDtypeStruct((NC, NPAD, QC), jnp.float32)
    return pl.pallas_call(
        _prep1_body,
        grid=(nblk,),
        in_specs=[
            pl.BlockSpec((blk, DIN), lambda i: (i, 0)),
            pl.BlockSpec((blk, 1), lambda i: (i, 0)),
            pl.BlockSpec((blk, 1), lambda i: (i, 0)),
            pl.BlockSpec((DIN, DM), lambda i: (0, 0)),
        ],
        out_specs=[qspec, qspec, pl.BlockSpec((blk, 1), lambda i: (i, 0))],
        out_shape=[qshape, qshape, jax.ShapeDtypeStruct((N, 1), jnp.float32)],
    )(xm, dega, degb, W1)


def _cat_quarters(a_ref, b_ref):
    return jnp.concatenate([a_ref[0], a_ref[1], b_ref[0], b_ref[1]], axis=1)


def _prep2_body(a_ref, b_ref, dinv_ref, b1_ref, w2_ref, oa_ref, ob_ref):
    agg = _cat_quarters(a_ref, b_ref)
    dinv = dinv_ref[...]
    h = jnp.maximum(agg * dinv + b1_ref[...], 0.0)
    hw = jnp.dot(h, w2_ref[...], preferred_element_type=jnp.float32)
    _split_quarters(hw * dinv, oa_ref, ob_ref)


def _tc_prep2(agg_a, agg_b, dinv, b1, W2):
    blk = NPG
    nblk = N // blk
    qspec = pl.BlockSpec((NC, blk, QC), lambda i: (0, i, 0))
    qshape = jax.ShapeDtypeStruct((NC, NPAD, QC), jnp.float32)
    return pl.pallas_call(
        _prep2_body,
        grid=(nblk,),
        in_specs=[
            qspec,
            qspec,
            pl.BlockSpec((blk, 1), lambda i: (i, 0)),
            pl.BlockSpec((1, DM), lambda i: (0, 0)),
            pl.BlockSpec((DM, DM), lambda i: (0, 0)),
        ],
        out_specs=[qspec, qspec],
        out_shape=[qshape, qshape],
    )(agg_a, agg_b, dinv, b1, W2)


def _final_body(a_ref, b_ref, dinv_ref, b2_ref, wh_ref, bh_ref, o_ref):
    agg = _cat_quarters(a_ref, b_ref)
    h = jnp.maximum(agg * dinv_ref[0] + b2_ref[...], 0.0)
    pooled = jnp.mean(h, axis=0, keepdims=True)
    o_ref[0] = (
        jnp.dot(pooled, wh_ref[...], preferred_element_type=jnp.float32)
        + bh_ref[...]
    )


def _tc_final(agg_a, agg_b, dinv, b2, Wh, bh):
    qspec = pl.BlockSpec((NC, NPG, QC), lambda b: (0, b, 0))
    return pl.pallas_call(
        _final_body,
        grid=(B,),
        in_specs=[
            qspec,
            qspec,
            pl.BlockSpec((1, NPG, 1), lambda b: (b, 0, 0)),
            pl.BlockSpec((1, DM), lambda b: (0, 0)),
            pl.BlockSpec((DM, DOUT), lambda b: (0, 0)),
            pl.BlockSpec((1, DOUT), lambda b: (0, 0)),
        ],
        out_specs=pl.BlockSpec((1, 1, DOUT), lambda b: (b, 0, 0)),
        out_shape=jax.ShapeDtypeStruct((B, 1, DOUT), jnp.float32),
    )(agg_a, agg_b, dinv, b2, Wh, bh)


@jax.jit
def kernel(x, edge_index, W1, b1, W2, b2, Wh, bh):
    src = edge_index[0]
    dst = edge_index[1]
    npad = EPAD - E
    srcp = jnp.concatenate([src, jnp.zeros((npad,), src.dtype)])
    dstp = jnp.concatenate([dst, jnp.full((npad,), N, dst.dtype)])
    src2 = jnp.concatenate([srcp, srcp + NPAD]).reshape(NC * ROWS, 128)
    dst2 = dstp.reshape(ROWS, 128)

    xr = x.reshape(B, S, NPG * DIN)
    xm = _tc_mean(xr).reshape(N, DIN)

    degp = _deg_call(dst2)
    dega = degp[:N].reshape(N, 1)
    degb = degp[DEG_LEN:DEG_LEN + N].reshape(N, 1)

    hws1a, hws1b, dinv = _tc_prep1(xm, dega, degb, W1)
    agg1a = _agg_call(src2, dst2, hws1a.reshape(NC * NPAD, QC))
    agg1b = _agg_call(src2, dst2, hws1b.reshape(NC * NPAD, QC))
    hws2a, hws2b = _tc_prep2(
        agg1a.reshape(NC, NPAD, QC), agg1b.reshape(NC, NPAD, QC),
        dinv, b1.reshape(1, DM), W2)
    agg2a = _agg_call(src2, dst2, hws2a.reshape(NC * NPAD, QC))
    agg2b = _agg_call(src2, dst2, hws2b.reshape(NC * NPAD, QC))
    out = _tc_final(
        agg2a.reshape(NC, NPAD, QC), agg2b.reshape(NC, NPAD, QC),
        dinv.reshape(B, NPG, 1),
        b2.reshape(1, DM),
        Wh,
        bh.reshape(1, DOUT),
    )
    return out.reshape(B, DOUT)


# speedup vs baseline: 17.6913x; 1.5511x over previous
import jax
import jax.numpy as jnp
from jax import lax
from jax.experimental import pallas as pl
from jax.experimental.pallas import tpu as pltpu
from jax.experimental.pallas import tpu_sc as plsc

B, S, NPG = 10, 8, 5000
N = B * NPG
E = 800000
DIN, DM, DOUT = 3, 64, 2
QC = 16

NC, NS = 2, 16
EPAD = 819200
ROWS = EPAD // 128
ROWS_PC = ROWS // NC
ROWS_PS = ROWS // NS
SROWS = 80
MACRO = 16
NMACRO = SROWS // MACRO

NPAD = 51200
SUBQ = NPAD // NS
CHUNK = 320

DEG_LEN = 50176
DEG_PS = DEG_LEN // NS


def _mesh():
    return plsc.VectorSubcoreMesh(core_axis_name="c", subcore_axis_name="s")


def _deg_body(dst_hbm, out_hbm, dst_v, ones_v, stage_v, acc):
    cid = lax.axis_index("c")
    sid = lax.axis_index("s")

    for i in range(8):
        ones_v[pl.ds(i * 16, 16)] = jnp.full((16,), 1.0, jnp.float32)

    def _zero(i, _):
        stage_v[pl.ds(i * 16, 16)] = jnp.zeros((16,), jnp.float32)
        return 0

    lax.fori_loop(0, DEG_PS // 16, _zero, 0)
    pltpu.sync_copy(stage_v, acc.at[pl.ds(sid * DEG_PS, DEG_PS)])
    plsc.subcore_barrier()

    row0 = cid * ROWS_PC + sid * (ROWS_PC // NS)
    pltpu.sync_copy(dst_hbm.at[pl.ds(row0, ROWS_PC // NS)], dst_v)

    def _scat(i, _):
        pltpu.sync_copy(ones_v, acc.at[dst_v.at[i]], add=True)
        return 0

    lax.fori_loop(0, ROWS_PC // NS, _scat, 0)
    plsc.subcore_barrier()

    pltpu.sync_copy(acc.at[pl.ds(sid * DEG_PS, DEG_PS)], stage_v)
    pltpu.sync_copy(stage_v, out_hbm.at[pl.ds(cid * DEG_LEN + sid * DEG_PS, DEG_PS)])


_deg_call = pl.kernel(
    _deg_body,
    out_type=jax.ShapeDtypeStruct((NC * DEG_LEN,), jnp.float32),
    mesh=_mesh(),
    scratch_types=[
        pltpu.VMEM((ROWS_PC // NS, 128), jnp.int32),
        pltpu.VMEM((128,), jnp.float32),
        pltpu.VMEM((DEG_PS,), jnp.float32),
        pltpu.VMEM_SHARED((DEG_LEN,), jnp.float32),
    ],
    compiler_params=pltpu.CompilerParams(use_tc_tiling_on_sc=False),
    name="sc_degree",
)


def _agg_body(src_hbm, dst_hbm, tab_hbm, out_hbm,
              src_v, dst_v, msgs_v, stage_v, gsem, ssem, acc):
    cid = lax.axis_index("c")
    sid = lax.axis_index("s")

    for t in range(SUBQ // CHUNK):
        r0 = sid * SUBQ + t * CHUNK
        pltpu.sync_copy(tab_hbm.at[pl.ds(cid * NPAD + r0, CHUNK)], stage_v)
        pltpu.sync_copy(stage_v, acc.at[pl.ds(r0, CHUNK)])
    plsc.subcore_barrier()

    row_base = sid * ROWS_PS

    def _stage(h, _):
        pltpu.sync_copy(
            src_hbm.at[pl.ds(cid * ROWS + row_base + h * SROWS, SROWS)], src_v)
        pltpu.sync_copy(dst_hbm.at[pl.ds(row_base + h * SROWS, SROWS)], dst_v)

        def _macro(m, _):
            r = m * MACRO
            gs = [
                pltpu.async_copy(tab_hbm.at[src_v.at[r + j]], msgs_v.at[j], gsem)
                for j in range(MACRO)
            ]
            ss = []
            for j in range(MACRO):
                gs[j].wait()
                ss.append(pltpu.async_copy(
                    msgs_v.at[j], acc.at[dst_v.at[r + j]], ssem, add=True))
            for s in ss:
                s.wait()
            return 0

        lax.fori_loop(0, NMACRO, _macro, 0)
        return 0

    lax.fori_loop(0, ROWS_PS // SROWS, _stage, 0)
    plsc.subcore_barrier()

    for t in range(SUBQ // CHUNK):
        r0 = sid * SUBQ + t * CHUNK
        pltpu.sync_copy(acc.at[pl.ds(r0, CHUNK)], stage_v)
        pltpu.sync_copy(stage_v, out_hbm.at[pl.ds(cid * NPAD + r0, CHUNK)])


_agg_call = pl.kernel(
    _agg_body,
    out_type=jax.ShapeDtypeStruct((NC * NPAD, QC), jnp.float32),
    mesh=_mesh(),
    scratch_types=[
        pltpu.VMEM((SROWS, 128), jnp.int32),
        pltpu.VMEM((SROWS, 128), jnp.int32),
        pltpu.VMEM((MACRO, 128, QC), jnp.float32),
        pltpu.VMEM((CHUNK, QC), jnp.float32),
        pltpu.SemaphoreType.DMA,
        pltpu.SemaphoreType.DMA,
        pltpu.VMEM_SHARED((NPAD, QC), jnp.float32),
    ],
    compiler_params=pltpu.CompilerParams(use_tc_tiling_on_sc=False),
    name="sc_edge_agg",
)


def _mean_body(x_ref, o_ref):
    o_ref[...] = jnp.mean(x_ref[...], axis=1)


def _tc_mean(xr):
    return pl.pallas_call(
        _mean_body,
        out_shape=jax.ShapeDtypeStruct((B, NPG * DIN), jnp.float32),
    )(xr)


def _split_quarters(hws, a_ref, b_ref):
    a_ref[0] = hws[:, 0 * QC:1 * QC]
    a_ref[1] = hws[:, 1 * QC:2 * QC]
    b_ref[0] = hws[:, 2 * QC:3 * QC]
    b_ref[1] = hws[:, 3 * QC:4 * QC]


def _prep1_body(xm_ref, da_ref, db_ref, w1_ref, a_ref, b_ref, dinv_ref):
    deg = da_ref[...] + db_ref[...] + 1.0
    dinv = lax.rsqrt(deg)
    hw = jnp.dot(xm_ref[...], w1_ref[...], preferred_element_type=jnp.float32)
    _split_quarters(hw * dinv, a_ref, b_ref)
    dinv_ref[...] = dinv


def _tc_prep1(xm, dega, degb, W1):
    blk = NPG
    nblk = N // blk
    qspec = pl.BlockSpec((NC, blk, QC), lambda i: (0, i, 0))
    qshape = jax.ShapeDtypeStruct((NC, NPAD, QC), jnp.float32)
    return pl.pallas_call(
        _prep1_body,
        grid=(nblk,),
        in_specs=[
            pl.BlockSpec((blk, DIN), lambda i: (i, 0)),
            pl.BlockSpec((blk, 1), lambda i: (i, 0)),
            pl.BlockSpec((blk, 1), lambda i: (i, 0)),
            pl.BlockSpec((DIN, DM), lambda i: (0, 0)),
        ],
        out_specs=[qspec, qspec, pl.BlockSpec((blk, 1), lambda i: (i, 0))],
        out_shape=[qshape, qshape, jax.ShapeDtypeStruct((N, 1), jnp.float32)],
    )(xm, dega, degb, W1)


def _cat_quarters(a_ref, b_ref):
    return jnp.concatenate([a_ref[0], a_ref[1], b_ref[0], b_ref[1]], axis=1)


def _prep2_body(a_ref, b_ref, dinv_ref, b1_ref, w2_ref, oa_ref, ob_ref):
    agg = _cat_quarters(a_ref, b_ref)
    dinv = dinv_ref[...]
    h = jnp.maximum(agg * dinv + b1_ref[...], 0.0)
    hw = jnp.dot(h, w2_ref[...], preferred_element_type=jnp.float32)
    _split_quarters(hw * dinv, oa_ref, ob_ref)


def _tc_prep2(agg_a, agg_b, dinv, b1, W2):
    blk = NPG
    nblk = N // blk
    qspec = pl.BlockSpec((NC, blk, QC), lambda i: (0, i, 0))
    qshape = jax.ShapeDtypeStruct((NC, NPAD, QC), jnp.float32)
    return pl.pallas_call(
        _prep2_body,
        grid=(nblk,),
        in_specs=[
            qspec,
            qspec,
            pl.BlockSpec((blk, 1), lambda i: (i, 0)),
            pl.BlockSpec((1, DM), lambda i: (0, 0)),
            pl.BlockSpec((DM, DM), lambda i: (0, 0)),
        ],
        out_specs=[qspec, qspec],
        out_shape=[qshape, qshape],
    )(agg_a, agg_b, dinv, b1, W2)


def _final_body(a_ref, b_ref, dinv_ref, b2_ref, wh_ref, bh_ref, o_ref):
    agg = _cat_quarters(a_ref, b_ref)
    h = jnp.maximum(agg * dinv_ref[0] + b2_ref[...], 0.0)
    pooled = jnp.mean(h, axis=0, keepdims=True)
    o_ref[0] = (
        jnp.dot(pooled, wh_ref[...], preferred_element_type=jnp.float32)
        + bh_ref[...]
    )


def _tc_final(agg_a, agg_b, dinv, b2, Wh, bh):
    qspec = pl.BlockSpec((NC, NPG, QC), lambda b: (0, b, 0))
    return pl.pallas_call(
        _final_body,
        grid=(B,),
        in_specs=[
            qspec,
            qspec,
            pl.BlockSpec((1, NPG, 1), lambda b: (b, 0, 0)),
            pl.BlockSpec((1, DM), lambda b: (0, 0)),
            pl.BlockSpec((DM, DOUT), lambda b: (0, 0)),
            pl.BlockSpec((1, DOUT), lambda b: (0, 0)),
        ],
        out_specs=pl.BlockSpec((1, 1, DOUT), lambda b: (b, 0, 0)),
        out_shape=jax.ShapeDtypeStruct((B, 1, DOUT), jnp.float32),
    )(agg_a, agg_b, dinv, b2, Wh, bh)


@jax.jit
def kernel(x, edge_index, W1, b1, W2, b2, Wh, bh):
    src = edge_index[0]
    dst = edge_index[1]
    npad = EPAD - E
    srcp = jnp.concatenate([src, jnp.zeros((npad,), src.dtype)])
    dstp = jnp.concatenate([dst, jnp.full((npad,), N, dst.dtype)])
    src2 = jnp.concatenate([srcp, srcp + NPAD]).reshape(NC * ROWS, 128)
    dst2 = dstp.reshape(ROWS, 128)

    xr = x.reshape(B, S, NPG * DIN)
    xm = _tc_mean(xr).reshape(N, DIN)

    degp = _deg_call(dst2)
    dega = degp[:N].reshape(N, 1)
    degb = degp[DEG_LEN:DEG_LEN + N].reshape(N, 1)

    hws1a, hws1b, dinv = _tc_prep1(xm, dega, degb, W1)
    agg1a = _agg_call(src2, dst2, hws1a.reshape(NC * NPAD, QC))
    agg1b = _agg_call(src2, dst2, hws1b.reshape(NC * NPAD, QC))
    hws2a, hws2b = _tc_prep2(
        agg1a.reshape(NC, NPAD, QC), agg1b.reshape(NC, NPAD, QC),
        dinv, b1.reshape(1, DM), W2)
    agg2a = _agg_call(src2, dst2, hws2a.reshape(NC * NPAD, QC))
    agg2b = _agg_call(src2, dst2, hws2b.reshape(NC * NPAD, QC))
    out = _tc_final(
        agg2a.reshape(NC, NPAD, QC), agg2b.reshape(NC, NPAD, QC),
        dinv.reshape(B, NPG, 1),
        b2.reshape(1, DM),
        Wh,
        bh.reshape(1, DOUT),
    )
    return out.reshape(B, DOUT)


# speedup vs baseline: 22.3902x; 1.2656x over previous
import jax
import jax.numpy as jnp
from jax import lax
from jax.experimental import pallas as pl
from jax.experimental.pallas import tpu as pltpu
from jax.experimental.pallas import tpu_sc as plsc

B, S, NPG = 10, 8, 5000
N = B * NPG
E = 800000
DIN, DM, DOUT = 3, 64, 2
QC = 16

NC, NS = 2, 16
EPAD = 819200
ROWS = EPAD // 128
ROWS_PC = ROWS // NC
ROWS_PS = ROWS // NS
SROWS = 80
MACRO = 16
NMACRO = SROWS // MACRO

NPAD = 51200
SUBQ = NPAD // NS
CHUNK = 320

DEG_LEN = 50176
DEG_PS = DEG_LEN // NS

XC = 16
A_SROWS = 100
A_MACRO = 10
A_NM = A_SROWS // A_MACRO
A_CHUNK = 320


def _mesh():
    return plsc.VectorSubcoreMesh(core_axis_name="c", subcore_axis_name="s")


def _deg_body(dst_hbm, out_hbm, dst_v, ones_v, stage_v, acc):
    cid = lax.axis_index("c")
    sid = lax.axis_index("s")

    for i in range(8):
        ones_v[pl.ds(i * 16, 16)] = jnp.full((16,), 1.0, jnp.float32)

    def _zero(i, _):
        stage_v[pl.ds(i * 16, 16)] = jnp.zeros((16,), jnp.float32)
        return 0

    lax.fori_loop(0, DEG_PS // 16, _zero, 0)
    pltpu.sync_copy(stage_v, acc.at[pl.ds(sid * DEG_PS, DEG_PS)])
    plsc.subcore_barrier()

    row0 = cid * ROWS_PC + sid * (ROWS_PC // NS)
    pltpu.sync_copy(dst_hbm.at[pl.ds(row0, ROWS_PC // NS)], dst_v)

    def _scat(i, _):
        pltpu.sync_copy(ones_v, acc.at[dst_v.at[i]], add=True)
        return 0

    lax.fori_loop(0, ROWS_PC // NS, _scat, 0)
    plsc.subcore_barrier()

    pltpu.sync_copy(acc.at[pl.ds(sid * DEG_PS, DEG_PS)], stage_v)
    pltpu.sync_copy(stage_v, out_hbm.at[pl.ds(cid * DEG_LEN + sid * DEG_PS, DEG_PS)])


_deg_call = pl.kernel(
    _deg_body,
    out_type=jax.ShapeDtypeStruct((NC * DEG_LEN,), jnp.float32),
    mesh=_mesh(),
    scratch_types=[
        pltpu.VMEM((ROWS_PC // NS, 128), jnp.int32),
        pltpu.VMEM((128,), jnp.float32),
        pltpu.VMEM((DEG_PS,), jnp.float32),
        pltpu.VMEM_SHARED((DEG_LEN,), jnp.float32),
    ],
    compiler_params=pltpu.CompilerParams(use_tc_tiling_on_sc=False),
    name="sc_degree",
)


def _aggx_body(src_hbm, dst_hbm, tabz_hbm, out_hbm,
               src_v, dst_v, msgs_v, stage_v, gsem, ssem, acc):
    cid = lax.axis_index("c")
    sid = lax.axis_index("s")

    for t in range(SUBQ // A_CHUNK):
        r0 = sid * SUBQ + t * A_CHUNK
        pltpu.sync_copy(tabz_hbm.at[pl.ds(cid * NPAD + r0, A_CHUNK)], stage_v)
        pltpu.sync_copy(stage_v, acc.at[pl.ds(r0, A_CHUNK)])
    plsc.subcore_barrier()

    base = cid * ROWS_PC + sid * (ROWS_PC // NS)

    def _stage(h, _):
        pltpu.sync_copy(src_hbm.at[pl.ds(base + h * A_SROWS, A_SROWS)], src_v)
        pltpu.sync_copy(dst_hbm.at[pl.ds(base + h * A_SROWS, A_SROWS)], dst_v)

        def _macro(m, _):
            r = m * A_MACRO
            gs = [
                pltpu.async_copy(tabz_hbm.at[src_v.at[r + j]], msgs_v.at[j], gsem)
                for j in range(A_MACRO)
            ]
            ss = []
            for j in range(A_MACRO):
                gs[j].wait()
                ss.append(pltpu.async_copy(
                    msgs_v.at[j], acc.at[dst_v.at[r + j]], ssem, add=True))
            for s in ss:
                s.wait()
            return 0

        lax.fori_loop(0, A_NM, _macro, 0)
        return 0

    lax.fori_loop(0, (ROWS_PC // NS) // A_SROWS, _stage, 0)
    plsc.subcore_barrier()

    for t in range(SUBQ // A_CHUNK):
        r0 = sid * SUBQ + t * A_CHUNK
        pltpu.sync_copy(acc.at[pl.ds(r0, A_CHUNK)], stage_v)
        pltpu.sync_copy(stage_v, out_hbm.at[pl.ds(cid * NPAD + r0, A_CHUNK)])


_aggx_call = pl.kernel(
    _aggx_body,
    out_type=jax.ShapeDtypeStruct((NC * NPAD, XC), jnp.float32),
    mesh=_mesh(),
    scratch_types=[
        pltpu.VMEM((A_SROWS, 128), jnp.int32),
        pltpu.VMEM((A_SROWS, 128), jnp.int32),
        pltpu.VMEM((A_MACRO, 128, XC), jnp.float32),
        pltpu.VMEM((A_CHUNK, XC), jnp.float32),
        pltpu.SemaphoreType.DMA,
        pltpu.SemaphoreType.DMA,
        pltpu.VMEM_SHARED((NPAD, XC), jnp.float32),
    ],
    compiler_params=pltpu.CompilerParams(use_tc_tiling_on_sc=False),
    name="sc_edge_aggx",
)


def _agg_body(src_hbm, dst_hbm, tab_hbm, out_hbm,
              src_v, dst_v, msgs_v, stage_v, gsem, ssem, acc):
    cid = lax.axis_index("c")
    sid = lax.axis_index("s")

    for t in range(SUBQ // CHUNK):
        r0 = sid * SUBQ + t * CHUNK
        pltpu.sync_copy(tab_hbm.at[pl.ds(cid * NPAD + r0, CHUNK)], stage_v)
        pltpu.sync_copy(stage_v, acc.at[pl.ds(r0, CHUNK)])
    plsc.subcore_barrier()

    row_base = sid * ROWS_PS

    def _stage(h, _):
        pltpu.sync_copy(
            src_hbm.at[pl.ds(cid * ROWS + row_base + h * SROWS, SROWS)], src_v)
        pltpu.sync_copy(dst_hbm.at[pl.ds(row_base + h * SROWS, SROWS)], dst_v)

        def _macro(m, _):
            r = m * MACRO
            gs = [
                pltpu.async_copy(tab_hbm.at[src_v.at[r + j]], msgs_v.at[j], gsem)
                for j in range(MACRO)
            ]
            ss = []
            for j in range(MACRO):
                gs[j].wait()
                ss.append(pltpu.async_copy(
                    msgs_v.at[j], acc.at[dst_v.at[r + j]], ssem, add=True))
            for s in ss:
                s.wait()
            return 0

        lax.fori_loop(0, NMACRO, _macro, 0)
        return 0

    lax.fori_loop(0, ROWS_PS // SROWS, _stage, 0)
    plsc.subcore_barrier()

    for t in range(SUBQ // CHUNK):
        r0 = sid * SUBQ + t * CHUNK
        pltpu.sync_copy(acc.at[pl.ds(r0, CHUNK)], stage_v)
        pltpu.sync_copy(stage_v, out_hbm.at[pl.ds(cid * NPAD + r0, CHUNK)])


_agg_call = pl.kernel(
    _agg_body,
    out_type=jax.ShapeDtypeStruct((NC * NPAD, QC), jnp.float32),
    mesh=_mesh(),
    scratch_types=[
        pltpu.VMEM((SROWS, 128), jnp.int32),
        pltpu.VMEM((SROWS, 128), jnp.int32),
        pltpu.VMEM((MACRO, 128, QC), jnp.float32),
        pltpu.VMEM((CHUNK, QC), jnp.float32),
        pltpu.SemaphoreType.DMA,
        pltpu.SemaphoreType.DMA,
        pltpu.VMEM_SHARED((NPAD, QC), jnp.float32),
    ],
    compiler_params=pltpu.CompilerParams(use_tc_tiling_on_sc=False),
    name="sc_edge_agg",
)


def _mean_body(x_ref, o_ref):
    o_ref[...] = jnp.mean(x_ref[...], axis=1)


def _tc_mean(xr):
    return pl.pallas_call(
        _mean_body,
        out_shape=jax.ShapeDtypeStruct((B, NPG * DIN), jnp.float32),
    )(xr)


def _split_quarters(hws, a_ref, b_ref):
    a_ref[0] = hws[:, 0 * QC:1 * QC]
    a_ref[1] = hws[:, 1 * QC:2 * QC]
    b_ref[0] = hws[:, 2 * QC:3 * QC]
    b_ref[1] = hws[:, 3 * QC:4 * QC]


def _prep1_body(xm_ref, da_ref, db_ref, xms_ref, dinv_ref):
    deg = da_ref[...] + db_ref[...] + 1.0
    dinv = lax.rsqrt(deg)
    xms = xm_ref[...] * dinv
    zpad = jnp.zeros((xms.shape[0], XC - DIN), jnp.float32)
    xms_ref[...] = jnp.concatenate([xms, zpad], axis=1)
    dinv_ref[...] = dinv


def _tc_prep1(xm, dega, degb):
    blk = NPG
    nblk = N // blk
    return pl.pallas_call(
        _prep1_body,
        grid=(nblk,),
        in_specs=[
            pl.BlockSpec((blk, DIN), lambda i: (i, 0)),
            pl.BlockSpec((blk, 1), lambda i: (i, 0)),
            pl.BlockSpec((blk, 1), lambda i: (i, 0)),
        ],
        out_specs=[
            pl.BlockSpec((blk, XC), lambda i: (i, 0)),
            pl.BlockSpec((blk, 1), lambda i: (i, 0)),
        ],
        out_shape=[
            jax.ShapeDtypeStruct((NPAD, XC), jnp.float32),
            jax.ShapeDtypeStruct((N, 1), jnp.float32),
        ],
    )(xm, dega, degb)


def _cat_quarters(a_ref, b_ref):
    return jnp.concatenate([a_ref[0], a_ref[1], b_ref[0], b_ref[1]], axis=1)


def _prep2_body(ax_ref, dinv_ref, w1_ref, b1_ref, w2_ref, oa_ref, ob_ref):
    aggx = (ax_ref[0] + ax_ref[1]) * dinv_ref[...]
    h = jnp.maximum(
        jnp.dot(aggx[:, :DIN], w1_ref[...], preferred_element_type=jnp.float32)
        + b1_ref[...], 0.0)
    hw = jnp.dot(h, w2_ref[...], preferred_element_type=jnp.float32)
    _split_quarters(hw * dinv_ref[...], oa_ref, ob_ref)


def _tc_prep2(aggx, dinv, W1, b1, W2):
    blk = NPG
    nblk = N // blk
    qspec = pl.BlockSpec((NC, blk, QC), lambda i: (0, i, 0))
    qshape = jax.ShapeDtypeStruct((NC, NPAD, QC), jnp.float32)
    return pl.pallas_call(
        _prep2_body,
        grid=(nblk,),
        in_specs=[
            pl.BlockSpec((NC, blk, XC), lambda i: (0, i, 0)),
            pl.BlockSpec((blk, 1), lambda i: (i, 0)),
            pl.BlockSpec((DIN, DM), lambda i: (0, 0)),
            pl.BlockSpec((1, DM), lambda i: (0, 0)),
            pl.BlockSpec((DM, DM), lambda i: (0, 0)),
        ],
        out_specs=[qspec, qspec],
        out_shape=[qshape, qshape],
    )(aggx, dinv, W1, b1, W2)


def _final_body(a_ref, b_ref, dinv_ref, b2_ref, wh_ref, bh_ref, o_ref):
    agg = _cat_quarters(a_ref, b_ref)
    h = jnp.maximum(agg * dinv_ref[0] + b2_ref[...], 0.0)
    pooled = jnp.mean(h, axis=0, keepdims=True)
    o_ref[0] = (
        jnp.dot(pooled, wh_ref[...], preferred_element_type=jnp.float32)
        + bh_ref[...]
    )


def _tc_final(agg_a, agg_b, dinv, b2, Wh, bh):
    qspec = pl.BlockSpec((NC, NPG, QC), lambda b: (0, b, 0))
    return pl.pallas_call(
        _final_body,
        grid=(B,),
        in_specs=[
            qspec,
            qspec,
            pl.BlockSpec((1, NPG, 1), lambda b: (b, 0, 0)),
            pl.BlockSpec((1, DM), lambda b: (0, 0)),
            pl.BlockSpec((DM, DOUT), lambda b: (0, 0)),
            pl.BlockSpec((1, DOUT), lambda b: (0, 0)),
        ],
        out_specs=pl.BlockSpec((1, 1, DOUT), lambda b: (b, 0, 0)),
        out_shape=jax.ShapeDtypeStruct((B, 1, DOUT), jnp.float32),
    )(agg_a, agg_b, dinv, b2, Wh, bh)


@jax.jit
def kernel(x, edge_index, W1, b1, W2, b2, Wh, bh):
    src = edge_index[0]
    dst = edge_index[1]
    npad = EPAD - E
    srcp = jnp.concatenate([src, jnp.zeros((npad,), src.dtype)])
    dstp = jnp.concatenate([dst, jnp.full((npad,), N, dst.dtype)])
    src1 = srcp.reshape(ROWS, 128)
    src2 = jnp.concatenate([srcp, srcp + NPAD]).reshape(NC * ROWS, 128)
    dst2 = dstp.reshape(ROWS, 128)

    xr = x.reshape(B, S, NPG * DIN)
    xm = _tc_mean(xr).reshape(N, DIN)

    degp = _deg_call(dst2)
    dega = degp[:N].reshape(N, 1)
    degb = degp[DEG_LEN:DEG_LEN + N].reshape(N, 1)

    xms, dinv = _tc_prep1(xm, dega, degb)
    tabz = jnp.concatenate([xms, jnp.zeros((NPAD, XC), jnp.float32)])
    aggx = _aggx_call(src1, dst2, tabz)
    hws2a, hws2b = _tc_prep2(
        aggx.reshape(NC, NPAD, XC), dinv, W1, b1.reshape(1, DM), W2)
    agg2a = _agg_call(src2, dst2, hws2a.reshape(NC * NPAD, QC))
    agg2b = _agg_call(src2, dst2, hws2b.reshape(NC * NPAD, QC))
    out = _tc_final(
        agg2a.reshape(NC, NPAD, QC), agg2b.reshape(NC, NPAD, QC),
        dinv.reshape(B, NPG, 1),
        b2.reshape(1, DM),
        Wh,
        bh.reshape(1, DOUT),
    )
    return out.reshape(B, DOUT)


# speedup vs baseline: 26.6387x; 1.1898x over previous
import jax
import jax.numpy as jnp
from jax import lax
from jax.experimental import pallas as pl
from jax.experimental.pallas import tpu as pltpu
from jax.experimental.pallas import tpu_sc as plsc

B, S, NPG = 10, 8, 5000
N = B * NPG
E = 800000
DIN, DM, DOUT = 3, 64, 2
QC = 16

NC, NS = 2, 16
EPAD = 819200
ROWS = EPAD // 128
ROWS_PC = ROWS // NC
ROWS_PS = ROWS // NS
SROWS = 80
MACRO = 16
NMACRO = SROWS // MACRO

NPAD = 51200
SUBQ = NPAD // NS
CHUNK = 320

DEG_LEN = 50176
DEG_PS = DEG_LEN // NS

XC = 16
A_SROWS = 100
A_MACRO = 10
A_NM = A_SROWS // A_MACRO
A_CHUNK = 320


def _mesh():
    return plsc.VectorSubcoreMesh(core_axis_name="c", subcore_axis_name="s")


def _deg_body(dst_hbm, out_hbm, dst_v, ones_v, stage_v, acc):
    cid = lax.axis_index("c")
    sid = lax.axis_index("s")

    for i in range(8):
        ones_v[pl.ds(i * 16, 16)] = jnp.full((16,), 1.0, jnp.float32)

    def _zero(i, _):
        stage_v[pl.ds(i * 16, 16)] = jnp.zeros((16,), jnp.float32)
        return 0

    lax.fori_loop(0, DEG_PS // 16, _zero, 0)
    pltpu.sync_copy(stage_v, acc.at[pl.ds(sid * DEG_PS, DEG_PS)])
    plsc.subcore_barrier()

    row0 = cid * ROWS_PC + sid * (ROWS_PC // NS)
    pltpu.sync_copy(dst_hbm.at[pl.ds(row0, ROWS_PC // NS)], dst_v)

    def _scat(i, _):
        pltpu.sync_copy(ones_v, acc.at[dst_v.at[i]], add=True)
        return 0

    lax.fori_loop(0, ROWS_PC // NS, _scat, 0)
    plsc.subcore_barrier()

    pltpu.sync_copy(acc.at[pl.ds(sid * DEG_PS, DEG_PS)], stage_v)
    pltpu.sync_copy(stage_v, out_hbm.at[pl.ds(cid * DEG_LEN + sid * DEG_PS, DEG_PS)])


_deg_call = pl.kernel(
    _deg_body,
    out_type=jax.ShapeDtypeStruct((NC * DEG_LEN,), jnp.float32),
    mesh=_mesh(),
    scratch_types=[
        pltpu.VMEM((ROWS_PC // NS, 128), jnp.int32),
        pltpu.VMEM((128,), jnp.float32),
        pltpu.VMEM((DEG_PS,), jnp.float32),
        pltpu.VMEM_SHARED((DEG_LEN,), jnp.float32),
    ],
    compiler_params=pltpu.CompilerParams(use_tc_tiling_on_sc=False),
    name="sc_degree",
)


def _aggx_body(src_hbm, dst_hbm, tabz_hbm, out_hbm,
               src_v, dst_v, msgs_v, stage_v, gsem, ssem, acc):
    cid = lax.axis_index("c")
    sid = lax.axis_index("s")

    @pl.when(cid == 0)
    def _init_tab():
        for t in range(SUBQ // A_CHUNK):
            r0 = sid * SUBQ + t * A_CHUNK
            pltpu.sync_copy(tabz_hbm.at[pl.ds(r0, A_CHUNK)], stage_v)
            pltpu.sync_copy(stage_v, acc.at[pl.ds(r0, A_CHUNK)])

    @pl.when(cid == 1)
    def _init_zero():
        def _z(i, _):
            stage_v[i] = jnp.zeros((XC,), jnp.float32)
            return 0
        lax.fori_loop(0, A_CHUNK, _z, 0)
        for t in range(SUBQ // A_CHUNK):
            pltpu.sync_copy(stage_v, acc.at[pl.ds(sid * SUBQ + t * A_CHUNK, A_CHUNK)])

    plsc.subcore_barrier()

    base = cid * ROWS_PC + sid * (ROWS_PC // NS)

    def _stage(h, _):
        pltpu.sync_copy(src_hbm.at[pl.ds(base + h * A_SROWS, A_SROWS)], src_v)
        pltpu.sync_copy(dst_hbm.at[pl.ds(base + h * A_SROWS, A_SROWS)], dst_v)

        def _macro(m, _):
            r = m * A_MACRO
            gs = [
                pltpu.async_copy(tabz_hbm.at[src_v.at[r + j]], msgs_v.at[j], gsem)
                for j in range(A_MACRO)
            ]
            ss = []
            for j in range(A_MACRO):
                gs[j].wait()
                ss.append(pltpu.async_copy(
                    msgs_v.at[j], acc.at[dst_v.at[r + j]], ssem, add=True))
            for s in ss:
                s.wait()
            return 0

        lax.fori_loop(0, A_NM, _macro, 0)
        return 0

    lax.fori_loop(0, (ROWS_PC // NS) // A_SROWS, _stage, 0)
    plsc.subcore_barrier()

    for t in range(SUBQ // A_CHUNK):
        r0 = sid * SUBQ + t * A_CHUNK
        pltpu.sync_copy(acc.at[pl.ds(r0, A_CHUNK)], stage_v)
        pltpu.sync_copy(stage_v, out_hbm.at[pl.ds(cid * NPAD + r0, A_CHUNK)])


_aggx_call = pl.kernel(
    _aggx_body,
    out_type=jax.ShapeDtypeStruct((NC * NPAD, XC), jnp.float32),
    mesh=_mesh(),
    scratch_types=[
        pltpu.VMEM((A_SROWS, 128), jnp.int32),
        pltpu.VMEM((A_SROWS, 128), jnp.int32),
        pltpu.VMEM((A_MACRO, 128, XC), jnp.float32),
        pltpu.VMEM((A_CHUNK, XC), jnp.float32),
        pltpu.SemaphoreType.DMA,
        pltpu.SemaphoreType.DMA,
        pltpu.VMEM_SHARED((NPAD, XC), jnp.float32),
    ],
    compiler_params=pltpu.CompilerParams(use_tc_tiling_on_sc=False),
    name="sc_edge_aggx",
)


def _agg_body(src_hbm, dst_hbm, tab_hbm, out_hbm,
              src_v, dst_v, msgs_v, stage_v, gsem, ssem, acc):
    cid = lax.axis_index("c")
    sid = lax.axis_index("s")

    for t in range(SUBQ // CHUNK):
        r0 = sid * SUBQ + t * CHUNK
        pltpu.sync_copy(tab_hbm.at[pl.ds(cid * NPAD + r0, CHUNK)], stage_v)
        pltpu.sync_copy(stage_v, acc.at[pl.ds(r0, CHUNK)])
    plsc.subcore_barrier()

    row_base = sid * ROWS_PS

    def _stage(h, _):
        pltpu.sync_copy(
            src_hbm.at[pl.ds(cid * ROWS + row_base + h * SROWS, SROWS)], src_v)
        pltpu.sync_copy(dst_hbm.at[pl.ds(row_base + h * SROWS, SROWS)], dst_v)

        def _macro(m, _):
            r = m * MACRO
            gs = [
                pltpu.async_copy(tab_hbm.at[src_v.at[r + j]], msgs_v.at[j], gsem)
                for j in range(MACRO)
            ]
            ss = []
            for j in range(MACRO):
                gs[j].wait()
                ss.append(pltpu.async_copy(
                    msgs_v.at[j], acc.at[dst_v.at[r + j]], ssem, add=True))
            for s in ss:
                s.wait()
            return 0

        lax.fori_loop(0, NMACRO, _macro, 0)
        return 0

    lax.fori_loop(0, ROWS_PS // SROWS, _stage, 0)
    plsc.subcore_barrier()

    for t in range(SUBQ // CHUNK):
        r0 = sid * SUBQ + t * CHUNK
        pltpu.sync_copy(acc.at[pl.ds(r0, CHUNK)], stage_v)
        pltpu.sync_copy(stage_v, out_hbm.at[pl.ds(cid * NPAD + r0, CHUNK)])


_agg_call = pl.kernel(
    _agg_body,
    out_type=jax.ShapeDtypeStruct((NC * NPAD, QC), jnp.float32),
    mesh=_mesh(),
    scratch_types=[
        pltpu.VMEM((SROWS, 128), jnp.int32),
        pltpu.VMEM((SROWS, 128), jnp.int32),
        pltpu.VMEM((MACRO, 128, QC), jnp.float32),
        pltpu.VMEM((CHUNK, QC), jnp.float32),
        pltpu.SemaphoreType.DMA,
        pltpu.SemaphoreType.DMA,
        pltpu.VMEM_SHARED((NPAD, QC), jnp.float32),
    ],
    compiler_params=pltpu.CompilerParams(use_tc_tiling_on_sc=False),
    name="sc_edge_agg",
)


def _mean_body(x_ref, o_ref):
    o_ref[...] = jnp.mean(x_ref[...], axis=2)


def _tc_mean(xT):
    return pl.pallas_call(
        _mean_body,
        grid=(B,),
        in_specs=[pl.BlockSpec((1, DIN, S, NPG), lambda b: (b, 0, 0, 0))],
        out_specs=pl.BlockSpec((1, DIN, NPG), lambda b: (b, 0, 0)),
        out_shape=jax.ShapeDtypeStruct((B, DIN, NPG), jnp.float32),
    )(xT)


def _split_quarters(hws, a_ref, b_ref):
    a_ref[0] = hws[:, 0 * QC:1 * QC]
    a_ref[1] = hws[:, 1 * QC:2 * QC]
    b_ref[0] = hws[:, 2 * QC:3 * QC]
    b_ref[1] = hws[:, 3 * QC:4 * QC]


def _prep1_body(xmT_ref, da_ref, db_ref, xms_ref, dinv_ref):
    deg = da_ref[0] + db_ref[0] + 1.0
    dinv = lax.rsqrt(deg)
    xms3 = xmT_ref[0] * dinv
    xmst = jnp.transpose(xms3)
    zpad = jnp.zeros((NPG, XC - DIN), jnp.float32)
    xms_ref[...] = jnp.concatenate([xmst, zpad], axis=1)
    dinv_ref[0] = dinv


def _tc_prep1(xmT, dega, degb):
    return pl.pallas_call(
        _prep1_body,
        grid=(B,),
        in_specs=[
            pl.BlockSpec((1, DIN, NPG), lambda i: (i, 0, 0)),
            pl.BlockSpec((1, 1, NPG), lambda i: (i, 0, 0)),
            pl.BlockSpec((1, 1, NPG), lambda i: (i, 0, 0)),
        ],
        out_specs=[
            pl.BlockSpec((NPG, XC), lambda i: (i, 0)),
            pl.BlockSpec((1, 1, NPG), lambda i: (i, 0, 0)),
        ],
        out_shape=[
            jax.ShapeDtypeStruct((NPAD, XC), jnp.float32),
            jax.ShapeDtypeStruct((B, 1, NPG), jnp.float32),
        ],
    )(xmT, dega, degb)


def _cat_quarters(a_ref, b_ref):
    return jnp.concatenate([a_ref[0], a_ref[1], b_ref[0], b_ref[1]], axis=1)


def _prep2_body(ax_ref, dinv_ref, w1_ref, b1_ref, w2_ref, oa_ref, ob_ref):
    dinv = jnp.transpose(dinv_ref[0])
    aggx = (ax_ref[0] + ax_ref[1]) * dinv
    h = jnp.maximum(
        jnp.dot(aggx[:, :DIN], w1_ref[...], preferred_element_type=jnp.float32)
        + b1_ref[...], 0.0)
    hw = jnp.dot(h, w2_ref[...], preferred_element_type=jnp.float32)
    _split_quarters(hw * dinv, oa_ref, ob_ref)


def _tc_prep2(aggx, dinv, W1, b1, W2):
    blk = NPG
    nblk = N // blk
    qspec = pl.BlockSpec((NC, blk, QC), lambda i: (0, i, 0))
    qshape = jax.ShapeDtypeStruct((NC, NPAD, QC), jnp.float32)
    return pl.pallas_call(
        _prep2_body,
        grid=(nblk,),
        in_specs=[
            pl.BlockSpec((NC, blk, XC), lambda i: (0, i, 0)),
            pl.BlockSpec((1, 1, NPG), lambda i: (i, 0, 0)),
            pl.BlockSpec((DIN, DM), lambda i: (0, 0)),
            pl.BlockSpec((1, DM), lambda i: (0, 0)),
            pl.BlockSpec((DM, DM), lambda i: (0, 0)),
        ],
        out_specs=[qspec, qspec],
        out_shape=[qshape, qshape],
    )(aggx, dinv, W1, b1, W2)


def _final_body(a_ref, b_ref, dinv_ref, b2_ref, wh_ref, bh_ref, o_ref):
    agg = _cat_quarters(a_ref, b_ref)
    dinv = jnp.transpose(dinv_ref[0])
    h = jnp.maximum(agg * dinv + b2_ref[...], 0.0)
    pooled = jnp.mean(h, axis=0, keepdims=True)
    o_ref[0] = (
        jnp.dot(pooled, wh_ref[...], preferred_element_type=jnp.float32)
        + bh_ref[...]
    )


def _tc_final(agg_a, agg_b, dinv, b2, Wh, bh):
    qspec = pl.BlockSpec((NC, NPG, QC), lambda b: (0, b, 0))
    return pl.pallas_call(
        _final_body,
        grid=(B,),
        in_specs=[
            qspec,
            qspec,
            pl.BlockSpec((1, 1, NPG), lambda b: (b, 0, 0)),
            pl.BlockSpec((1, DM), lambda b: (0, 0)),
            pl.BlockSpec((DM, DOUT), lambda b: (0, 0)),
            pl.BlockSpec((1, DOUT), lambda b: (0, 0)),
        ],
        out_specs=pl.BlockSpec((1, 1, DOUT), lambda b: (b, 0, 0)),
        out_shape=jax.ShapeDtypeStruct((B, 1, DOUT), jnp.float32),
    )(agg_a, agg_b, dinv, b2, Wh, bh)


@jax.jit
def kernel(x, edge_index, W1, b1, W2, b2, Wh, bh):
    src = edge_index[0]
    dst = edge_index[1]
    npad = EPAD - E
    srcp = jnp.concatenate([src, jnp.zeros((npad,), src.dtype)])
    dstp = jnp.concatenate([dst, jnp.full((npad,), N, dst.dtype)])
    src1 = srcp.reshape(ROWS, 128)
    src2 = jnp.concatenate([srcp, srcp + NPAD]).reshape(NC * ROWS, 128)
    dst2 = dstp.reshape(ROWS, 128)

    xT = x.transpose(0, 3, 1, 2)
    xmT = _tc_mean(xT)

    degp = _deg_call(dst2)
    dega = degp[:N].reshape(B, 1, NPG)
    degb = degp[DEG_LEN:DEG_LEN + N].reshape(B, 1, NPG)

    xms, dinv = _tc_prep1(xmT, dega, degb)
    aggx = _aggx_call(src1, dst2, xms)
    hws2a, hws2b = _tc_prep2(
        aggx.reshape(NC, NPAD, XC), dinv, W1, b1.reshape(1, DM), W2)
    agg2a = _agg_call(src2, dst2, hws2a.reshape(NC * NPAD, QC))
    agg2b = _agg_call(src2, dst2, hws2b.reshape(NC * NPAD, QC))
    out = _tc_final(
        agg2a.reshape(NC, NPAD, QC), agg2b.reshape(NC, NPAD, QC),
        dinv,
        b2.reshape(1, DM),
        Wh,
        bh.reshape(1, DOUT),
    )
    return out.reshape(B, DOUT)


# speedup vs baseline: 27.9333x; 1.0486x over previous
import jax
import jax.numpy as jnp
from jax import lax
from jax.experimental import pallas as pl
from jax.experimental.pallas import tpu as pltpu
from jax.experimental.pallas import tpu_sc as plsc

B, S, NPG = 10, 8, 5000
N = B * NPG
E = 800000
DIN, DM, DOUT = 3, 64, 2
QC = 16

NC, NS = 2, 16
EPAD = 819200
ROWS = EPAD // 128
ROWS_PC = ROWS // NC
ROWS_PS = ROWS // NS
SROWS = 80
MACRO = 16
NMACRO = SROWS // MACRO

NPAD = 51200
SUBQ = NPAD // NS
CHUNK = 400
NCH = SUBQ // CHUNK
NCH_LAST = (N - (NS - 1) * SUBQ) // CHUNK
PKB = NPG * QC // 128

DEG_LEN = 50176
DEG_PS = DEG_LEN // NS

XC = 16
A_SROWS = 100
A_MACRO = 10
A_NM = A_SROWS // A_MACRO
A_CHUNK = 400


def _mesh():
    return plsc.VectorSubcoreMesh(core_axis_name="c", subcore_axis_name="s")


def _deg_body(dst_hbm, out_hbm, dst_v, ones_v, stage_v, acc):
    cid = lax.axis_index("c")
    sid = lax.axis_index("s")

    for i in range(8):
        ones_v[pl.ds(i * 16, 16)] = jnp.full((16,), 1.0, jnp.float32)

    def _zero(i, _):
        stage_v[pl.ds(i * 16, 16)] = jnp.zeros((16,), jnp.float32)
        return 0

    lax.fori_loop(0, DEG_PS // 16, _zero, 0)
    pltpu.sync_copy(stage_v, acc.at[pl.ds(sid * DEG_PS, DEG_PS)])
    plsc.subcore_barrier()

    row0 = cid * ROWS_PC + sid * (ROWS_PC // NS)
    pltpu.sync_copy(dst_hbm.at[pl.ds(row0, ROWS_PC // NS)], dst_v)

    def _scat(i, _):
        pltpu.sync_copy(ones_v, acc.at[dst_v.at[i]], add=True)
        return 0

    lax.fori_loop(0, ROWS_PC // NS, _scat, 0)
    plsc.subcore_barrier()

    pltpu.sync_copy(acc.at[pl.ds(sid * DEG_PS, DEG_PS)], stage_v)
    pltpu.sync_copy(stage_v, out_hbm.at[pl.ds(cid * DEG_LEN + sid * DEG_PS, DEG_PS)])


_deg_call = pl.kernel(
    _deg_body,
    out_type=jax.ShapeDtypeStruct((NC * DEG_LEN,), jnp.float32),
    mesh=_mesh(),
    scratch_types=[
        pltpu.VMEM((ROWS_PC // NS, 128), jnp.int32),
        pltpu.VMEM((128,), jnp.float32),
        pltpu.VMEM((DEG_PS,), jnp.float32),
        pltpu.VMEM_SHARED((DEG_LEN,), jnp.float32),
    ],
    compiler_params=pltpu.CompilerParams(use_tc_tiling_on_sc=False),
    name="sc_degree",
)


def _aggx_body(src_hbm, dst_hbm, tabz_hbm, out_hbm,
               src_v, dst_v, msgs_v, stage_v, gsem, ssem, acc):
    cid = lax.axis_index("c")
    sid = lax.axis_index("s")

    nch = jnp.where(sid == NS - 1, NCH_LAST, NCH)

    @pl.when(cid == 0)
    def _init_tab():
        def _cp(t, _):
            r0 = sid * SUBQ + t * A_CHUNK
            pltpu.sync_copy(tabz_hbm.at[pl.ds(r0, A_CHUNK)], stage_v)
            pltpu.sync_copy(stage_v, acc.at[pl.ds(r0, A_CHUNK)])
            return 0
        lax.fori_loop(0, nch, _cp, 0)

    @pl.when(cid == 1)
    def _init_zero():
        def _z(i, _):
            stage_v[i] = jnp.zeros((XC,), jnp.float32)
            return 0
        lax.fori_loop(0, A_CHUNK, _z, 0)
        def _zc(t, _):
            pltpu.sync_copy(stage_v, acc.at[pl.ds(sid * SUBQ + t * A_CHUNK, A_CHUNK)])
            return 0
        lax.fori_loop(0, nch, _zc, 0)

    plsc.subcore_barrier()

    base = cid * ROWS_PC + sid * (ROWS_PC // NS)

    def _stage(h, _):
        pltpu.sync_copy(src_hbm.at[pl.ds(base + h * A_SROWS, A_SROWS)], src_v)
        pltpu.sync_copy(dst_hbm.at[pl.ds(base + h * A_SROWS, A_SROWS)], dst_v)

        def _macro(m, _):
            r = m * A_MACRO
            gs = [
                pltpu.async_copy(tabz_hbm.at[src_v.at[r + j]], msgs_v.at[j], gsem)
                for j in range(A_MACRO)
            ]
            ss = []
            for j in range(A_MACRO):
                gs[j].wait()
                ss.append(pltpu.async_copy(
                    msgs_v.at[j], acc.at[dst_v.at[r + j]], ssem, add=True))
            for s in ss:
                s.wait()
            return 0

        lax.fori_loop(0, A_NM, _macro, 0)
        return 0

    lax.fori_loop(0, (ROWS_PC // NS) // A_SROWS, _stage, 0)
    plsc.subcore_barrier()

    def _drain(t, _):
        r0 = sid * SUBQ + t * A_CHUNK
        pltpu.sync_copy(acc.at[pl.ds(r0, A_CHUNK)], stage_v)
        pltpu.sync_copy(stage_v, out_hbm.at[pl.ds(cid * N + r0, A_CHUNK)])
        return 0
    lax.fori_loop(0, nch, _drain, 0)


_aggx_call = pl.kernel(
    _aggx_body,
    out_type=jax.ShapeDtypeStruct((NC * N, XC), jnp.float32),
    mesh=_mesh(),
    scratch_types=[
        pltpu.VMEM((A_SROWS, 128), jnp.int32),
        pltpu.VMEM((A_SROWS, 128), jnp.int32),
        pltpu.VMEM((A_MACRO, 128, XC), jnp.float32),
        pltpu.VMEM((A_CHUNK, XC), jnp.float32),
        pltpu.SemaphoreType.DMA,
        pltpu.SemaphoreType.DMA,
        pltpu.VMEM_SHARED((NPAD, XC), jnp.float32),
    ],
    compiler_params=pltpu.CompilerParams(use_tc_tiling_on_sc=False),
    name="sc_edge_aggx",
)


def _agg_body(src_hbm, dst_hbm, tab_hbm, out_hbm,
              src_v, dst_v, msgs_v, stage_v, gsem, ssem, acc):
    cid = lax.axis_index("c")
    sid = lax.axis_index("s")
    nch = jnp.where(sid == NS - 1, NCH_LAST, NCH)

    def _init(t, _):
        r0 = sid * SUBQ + t * CHUNK
        pltpu.sync_copy(tab_hbm.at[pl.ds(cid * N + r0, CHUNK)], stage_v)
        pltpu.sync_copy(stage_v, acc.at[pl.ds(r0, CHUNK)])
        return 0
    lax.fori_loop(0, nch, _init, 0)
    plsc.subcore_barrier()

    row_base = sid * ROWS_PS

    def _stage(h, _):
        pltpu.sync_copy(
            src_hbm.at[pl.ds(cid * ROWS + row_base + h * SROWS, SROWS)], src_v)
        pltpu.sync_copy(dst_hbm.at[pl.ds(row_base + h * SROWS, SROWS)], dst_v)

        def _macro(m, _):
            r = m * MACRO
            gs = [
                pltpu.async_copy(tab_hbm.at[src_v.at[r + j]], msgs_v.at[j], gsem)
                for j in range(MACRO)
            ]
            ss = []
            for j in range(MACRO):
                gs[j].wait()
                ss.append(pltpu.async_copy(
                    msgs_v.at[j], acc.at[dst_v.at[r + j]], ssem, add=True))
            for s in ss:
                s.wait()
            return 0

        lax.fori_loop(0, NMACRO, _macro, 0)
        return 0

    lax.fori_loop(0, ROWS_PS // SROWS, _stage, 0)
    plsc.subcore_barrier()

    def _drain(t, _):
        r0 = sid * SUBQ + t * CHUNK
        pltpu.sync_copy(acc.at[pl.ds(r0, CHUNK)], stage_v)
        pltpu.sync_copy(stage_v, out_hbm.at[pl.ds(cid * N + r0, CHUNK)])
        return 0
    lax.fori_loop(0, nch, _drain, 0)


_agg_call = pl.kernel(
    _agg_body,
    out_type=jax.ShapeDtypeStruct((NC * N, QC), jnp.float32),
    mesh=_mesh(),
    scratch_types=[
        pltpu.VMEM((SROWS, 128), jnp.int32),
        pltpu.VMEM((SROWS, 128), jnp.int32),
        pltpu.VMEM((MACRO, 128, QC), jnp.float32),
        pltpu.VMEM((CHUNK, QC), jnp.float32),
        pltpu.SemaphoreType.DMA,
        pltpu.SemaphoreType.DMA,
        pltpu.VMEM_SHARED((NPAD, QC), jnp.float32),
    ],
    compiler_params=pltpu.CompilerParams(use_tc_tiling_on_sc=False),
    name="sc_edge_agg",
)


def _mean_body(x_ref, o_ref):
    o_ref[...] = jnp.mean(x_ref[...], axis=2)


def _tc_mean(xT):
    return pl.pallas_call(
        _mean_body,
        grid=(B,),
        in_specs=[pl.BlockSpec((1, DIN, S, NPG), lambda b: (b, 0, 0, 0))],
        out_specs=pl.BlockSpec((1, DIN, NPG), lambda b: (b, 0, 0)),
        out_shape=jax.ShapeDtypeStruct((B, DIN, NPG), jnp.float32),
    )(xT)


def _split_quarters(hws, a_ref, b_ref):
    a_ref[0] = hws[:, 0 * QC:1 * QC]
    a_ref[1] = hws[:, 1 * QC:2 * QC]
    b_ref[0] = hws[:, 2 * QC:3 * QC]
    b_ref[1] = hws[:, 3 * QC:4 * QC]


def _prep1_body(xmT_ref, da_ref, db_ref, xms_ref, dinv_ref):
    deg = da_ref[0] + db_ref[0] + 1.0
    dinv = lax.rsqrt(deg)
    xms3 = xmT_ref[0] * dinv
    xmst = jnp.transpose(xms3)
    zpad = jnp.zeros((NPG, XC - DIN), jnp.float32)
    xms = jnp.concatenate([xmst, zpad], axis=1)
    xms_ref[...] = xms
    dinv_ref[0] = dinv


def _tc_prep1(xmT, dega, degb):
    return pl.pallas_call(
        _prep1_body,
        grid=(B,),
        in_specs=[
            pl.BlockSpec((1, DIN, NPG), lambda i: (i, 0, 0)),
            pl.BlockSpec((1, 1, NPG), lambda i: (i, 0, 0)),
            pl.BlockSpec((1, 1, NPG), lambda i: (i, 0, 0)),
        ],
        out_specs=[
            pl.BlockSpec((NPG, XC), lambda i: (i, 0)),
            pl.BlockSpec((1, 1, NPG), lambda i: (i, 0, 0)),
        ],
        out_shape=[
            jax.ShapeDtypeStruct((N, XC), jnp.float32),
            jax.ShapeDtypeStruct((B, 1, NPG), jnp.float32),
        ],
    )(xmT, dega, degb)


def _cat_quarters(a_ref, b_ref):
    return jnp.concatenate([a_ref[0], a_ref[1], b_ref[0], b_ref[1]], axis=1)


def _prep2_body(ax_ref, dinv_ref, w1_ref, b1_ref, w2_ref, oa_ref, ob_ref):
    dinv = jnp.transpose(dinv_ref[0])
    aggx = (ax_ref[0] + ax_ref[1]) * dinv
    h = jnp.maximum(
        jnp.dot(aggx[:, :DIN], w1_ref[...], preferred_element_type=jnp.float32)
        + b1_ref[...], 0.0)
    hw = jnp.dot(h, w2_ref[...], preferred_element_type=jnp.float32)
    _split_quarters(hw * dinv, oa_ref, ob_ref)


def _tc_prep2(aggx, dinv, W1, b1, W2):
    blk = NPG
    nblk = N // blk
    qspec = pl.BlockSpec((NC, blk, QC), lambda i: (0, i, 0))
    qshape = jax.ShapeDtypeStruct((NC, N, QC), jnp.float32)
    return pl.pallas_call(
        _prep2_body,
        grid=(nblk,),
        in_specs=[
            pl.BlockSpec((NC, blk, XC), lambda i: (0, i, 0)),
            pl.BlockSpec((1, 1, NPG), lambda i: (i, 0, 0)),
            pl.BlockSpec((DIN, DM), lambda i: (0, 0)),
            pl.BlockSpec((1, DM), lambda i: (0, 0)),
            pl.BlockSpec((DM, DM), lambda i: (0, 0)),
        ],
        out_specs=[qspec, qspec],
        out_shape=[qshape, qshape],
    )(aggx, dinv, W1, b1, W2)


def _final_body(a_ref, b_ref, dinv_ref, b2_ref, wh_ref, bh_ref, o_ref):
    agg = _cat_quarters(a_ref, b_ref)
    dinv = jnp.transpose(dinv_ref[0])
    h = jnp.maximum(agg * dinv + b2_ref[...], 0.0)
    pooled = jnp.mean(h, axis=0, keepdims=True)
    o_ref[0] = (
        jnp.dot(pooled, wh_ref[...], preferred_element_type=jnp.float32)
        + bh_ref[...]
    )


def _tc_final(agg_a, agg_b, dinv, b2, Wh, bh):
    qspec = pl.BlockSpec((NC, NPG, QC), lambda b: (0, b, 0))
    return pl.pallas_call(
        _final_body,
        grid=(B,),
        in_specs=[
            qspec,
            qspec,
            pl.BlockSpec((1, 1, NPG), lambda b: (b, 0, 0)),
            pl.BlockSpec((1, DM), lambda b: (0, 0)),
            pl.BlockSpec((DM, DOUT), lambda b: (0, 0)),
            pl.BlockSpec((1, DOUT), lambda b: (0, 0)),
        ],
        out_specs=pl.BlockSpec((1, 1, DOUT), lambda b: (b, 0, 0)),
        out_shape=jax.ShapeDtypeStruct((B, 1, DOUT), jnp.float32),
    )(agg_a, agg_b, dinv, b2, Wh, bh)


@jax.jit
def kernel(x, edge_index, W1, b1, W2, b2, Wh, bh):
    src = edge_index[0]
    dst = edge_index[1]
    npad = EPAD - E
    srcp = jnp.concatenate([src, jnp.zeros((npad,), src.dtype)])
    dstp = jnp.concatenate([dst, jnp.full((npad,), N, dst.dtype)])
    src1 = srcp.reshape(ROWS, 128)
    src2 = jnp.concatenate([srcp, srcp + N]).reshape(NC * ROWS, 128)
    dst2 = dstp.reshape(ROWS, 128)

    xT = x.transpose(0, 3, 1, 2)
    xmT = _tc_mean(xT)

    degp = _deg_call(dst2)
    dega = degp[:N].reshape(B, 1, NPG)
    degb = degp[DEG_LEN:DEG_LEN + N].reshape(B, 1, NPG)

    xms, dinv = _tc_prep1(xmT, dega, degb)
    aggx = _aggx_call(src1, dst2, xms)
    hws2a, hws2b = _tc_prep2(
        aggx.reshape(NC, N, XC), dinv, W1, b1.reshape(1, DM), W2)
    agg2a = _agg_call(src2, dst2, hws2a.reshape(NC * N, QC))
    agg2b = _agg_call(src2, dst2, hws2b.reshape(NC * N, QC))
    out = _tc_final(
        agg2a.reshape(NC, N, QC),
        agg2b.reshape(NC, N, QC),
        dinv,
        b2.reshape(1, DM),
        Wh,
        bh.reshape(1, DOUT),
    )
    return out.reshape(B, DOUT)
